# baseline (speedup 1.0000x reference)
import functools

import jax
import jax.numpy as jnp
from jax import lax
from jax.experimental import pallas as pl
from jax.experimental.pallas import tpu as pltpu
from jax.experimental.pallas import tpu_sc as plsc

DIM = 16
LANES = 16
NC = 2
NS = 16
NW = NC * NS
CHUNK = 1024
NG = 2
NO = 2
TBLK = 1024
TUNROLL = 8


@functools.lru_cache(maxsize=None)
def _transpose_sc(vocab):
    nblk = vocab // TBLK
    rem = vocab - nblk * TBLK
    tail_b = (rem // 128) * 128
    per_w = (nblk + NW - 1) // NW
    mesh = plsc.VectorSubcoreMesh(core_axis_name="c", subcore_axis_name="s")

    @functools.partial(
        pl.kernel,
        mesh=mesh,
        compiler_params=pltpu.CompilerParams(
            needs_layout_passes=False, use_tc_tiling_on_sc=True
        ),
        out_type=jax.ShapeDtypeStruct((vocab * DIM,), jnp.float32),
        scratch_types=[
            pltpu.VMEM((DIM, TBLK), jnp.float32),
            pltpu.VMEM((DIM, TBLK), jnp.float32),
            pltpu.VMEM((TBLK * DIM,), jnp.float32),
            pltpu.VMEM((TBLK * DIM,), jnp.float32),
            pltpu.SemaphoreType.DMA,
            pltpu.SemaphoreType.DMA,
            pltpu.SemaphoreType.DMA,
            pltpu.SemaphoreType.DMA,
        ],
    )
    def k(projT_hbm, tail_hbm, flat_hbm, tb0, tb1, rb0, rb1, *sems):
        tbuf = (tb0, tb1)
        rbuf = (rb0, rb1)
        isem = sems[:2]
        osem = sems[2:]
        c = lax.axis_index("c")
        s = lax.axis_index("s")
        wid = s * NC + c
        iota = lax.iota(jnp.int32, LANES)

        def transpose_cols(tb, rb, ncols):
            unroll = 1

            def body(g, carry):
                for u in range(unroll):
                    i0 = (g * unroll + u) * LANES
                    base = (lax.broadcast(i0, (LANES,)) + iota) * DIM
                    for d in range(DIM):
                        v = tbuf[tb][d, pl.ds(i0, LANES)]
                        plsc.store_scatter(
                            rbuf[rb], [base + jnp.int32(d)], v
                        )
                return carry

            lax.fori_loop(0, ncols // (unroll * LANES), body, 0)

        def src_dst(t):
            blk = wid + t * NW
            b0 = pl.multiple_of(blk * TBLK, 128)
            src = projT_hbm.at[:, pl.ds(b0, TBLK)]
            dst = flat_hbm.at[pl.ds(b0 * DIM, TBLK * DIM)]
            return src, dst

        def guard(t):
            return wid + t * NW < nblk

        @pl.when(guard(0))
        def _():
            src, _ = src_dst(0)
            pltpu.async_copy(src, tbuf[0], isem[0])

        for t in range(per_w):
            p = t % 2

            @pl.when(guard(t))
            def _(t=t, p=p):
                src, _ = src_dst(t)
                pltpu.make_async_copy(src, tbuf[p], isem[p]).wait()

            if t + 1 < per_w:
                @pl.when(guard(t + 1))
                def _(t=t, p=p):
                    src, _ = src_dst(t + 1)
                    pltpu.async_copy(src, tbuf[1 - p], isem[1 - p])

            if t - 2 >= 0:
                @pl.when(guard(t))
                def _(t=t, p=p):
                    _, dst = src_dst(t - 2)
                    pltpu.make_async_copy(rbuf[p], dst, osem[p]).wait()

            @pl.when(guard(t))
            def _(t=t, p=p):
                transpose_cols(p, p, TBLK)
                _, dst = src_dst(t)
                pltpu.async_copy(rbuf[p], dst, osem[p])

        for t in range(max(0, per_w - 3), per_w):
            @pl.when(jnp.logical_and(guard(t), jnp.logical_not(guard(t + 2))))
            def _(t=t):
                _, dst = src_dst(t)
                pltpu.make_async_copy(rbuf[t % 2], dst, osem[t % 2]).wait()

        if tail_b:
            @pl.when(wid == NW - 1)
            def _():
                b0 = pl.multiple_of(nblk * TBLK, 128)
                pltpu.sync_copy(
                    projT_hbm.at[:, pl.ds(b0, tail_b)],
                    tbuf[0].at[:, pl.ds(0, tail_b)],
                )
                transpose_cols(0, 0, tail_b)
                pltpu.sync_copy(
                    rbuf[0].at[pl.ds(0, tail_b * DIM)],
                    flat_hbm.at[pl.ds(nblk * TBLK * DIM, tail_b * DIM)],
                )

        @pl.when(wid == NW - 2)
        def _():
            pltpu.sync_copy(tail_hbm, tbuf[0].at[:, pl.ds(0, 128)])
            transpose_cols(0, 0, 128)
            pltpu.sync_copy(
                rbuf[0].at[pl.ds(0, 128 * DIM)],
                flat_hbm.at[pl.ds((vocab - 128) * DIM, 128 * DIM)],
            )

    return k


@functools.lru_cache(maxsize=None)
def _softalign_sc(batch, fields):
    n_rows = batch * fields
    bpw = n_rows // NW
    nch = bpw // CHUNK
    cpf = batch // CHUNK
    mesh = plsc.VectorSubcoreMesh(core_axis_name="c", subcore_axis_name="s")

    @functools.partial(
        pl.kernel,
        mesh=mesh,
        compiler_params=pltpu.CompilerParams(
            needs_layout_passes=False, use_tc_tiling_on_sc=False
        ),
        out_type=jax.ShapeDtypeStruct((fields, DIM, batch), jnp.float32),
        scratch_types=(
            [
                pltpu.VMEM((nch, CHUNK), jnp.int32),
                pltpu.VMEM((NG, CHUNK, DIM), jnp.float32),
                pltpu.VMEM((NO, DIM, CHUNK), jnp.float32),
            ]
            + [pltpu.SemaphoreType.DMA] * (NG + NO)
        ),
    )
    def k(idxT_hbm, proj_hbm, outT_hbm, idx_v, gbuf_v, obuf_v, *sems):
        gsem = sems[:NG]
        osem = sems[NG:]

        c = lax.axis_index("c")
        s = lax.axis_index("s")
        wid = s * NC + c
        jc0 = wid * nch

        for t in range(nch):
            jc = jc0 + t
            f = jc // cpf
            b0 = (jc % cpf) * CHUNK
            pltpu.sync_copy(idxT_hbm.at[f, pl.ds(b0, CHUNK)], idx_v.at[t])

        def fire_gather(t):
            b = t % NG
            return pltpu.async_copy(
                proj_hbm.at[idx_v.at[t]], gbuf_v.at[b], gsem[b]
            )

        pending_g = {0: fire_gather(0)}
        pending_o = {}
        iota = lax.iota(jnp.int32, LANES)

        for j in range(nch):
            b = j % NG
            o = j % NO
            pending_g.pop(j).wait()
            if j + 1 < nch:
                pending_g[j + 1] = fire_gather(j + 1)
            if j - NO >= 0:
                pending_o.pop(j - NO).wait()

            def body(g, carry):
                for u in range(2):
                    g0 = g * 2 + u
                    rows = lax.broadcast(g0 * LANES, (LANES,)) + iota
                    es = []
                    for col in range(DIM):
                        cols = jnp.full((LANES,), col, jnp.int32)
                        v = plsc.load_gather(gbuf_v.at[b], [rows, cols])
                        es.append(jnp.exp(v))
                    acc = es
                    while len(acc) > 1:
                        acc = [
                            acc[i] + acc[i + 1] for i in range(0, len(acc), 2)
                        ]
                    r = 1.0 / acc[0]
                    for col in range(DIM):
                        obuf_v[o, col, pl.ds(g0 * LANES, LANES)] = es[col] * r
                return carry

            lax.fori_loop(0, CHUNK // LANES // 2, body, 0)

            jc = jc0 + j
            f = jc // cpf
            b0 = (jc % cpf) * CHUNK
            pending_o[j] = pltpu.async_copy(
                obuf_v.at[o],
                outT_hbm.at[f, :, pl.ds(b0, CHUNK)],
                osem[o],
            )

        for j in sorted(pending_o):
            pending_o[j].wait()

    return k


def kernel(input, proj):
    batch, fields = input.shape
    vocab = proj.shape[0]
    projT = proj.T
    tail128 = lax.slice(projT, (0, vocab - 128), (DIM, vocab))
    flat = _transpose_sc(vocab)(projT, tail128)
    proj_rm = flat.reshape(vocab, DIM)
    outT = _softalign_sc(batch, fields)(input.T.astype(jnp.int32), proj_rm)
    return outT.transpose(2, 0, 1)


# speedup vs baseline: 1.0010x; 1.0010x over previous
import functools

import jax
import jax.numpy as jnp
from jax import lax
from jax.experimental import pallas as pl
from jax.experimental.pallas import tpu as pltpu
from jax.experimental.pallas import tpu_sc as plsc

DIM = 16
LANES = 16
NC = 2
NS = 16
NW = NC * NS
CHUNK = 1024
NG = 2
NO = 2
TBLK = 1536
TUNROLL = 8


@functools.lru_cache(maxsize=None)
def _transpose_sc(vocab):
    nblk = vocab // TBLK
    rem = vocab - nblk * TBLK
    tail_b = (rem // 128) * 128
    per_w = (nblk + NW - 1) // NW
    mesh = plsc.VectorSubcoreMesh(core_axis_name="c", subcore_axis_name="s")

    @functools.partial(
        pl.kernel,
        mesh=mesh,
        compiler_params=pltpu.CompilerParams(
            needs_layout_passes=False, use_tc_tiling_on_sc=True
        ),
        out_type=jax.ShapeDtypeStruct((vocab * DIM,), jnp.float32),
        scratch_types=[
            pltpu.VMEM((DIM, TBLK), jnp.float32),
            pltpu.VMEM((DIM, TBLK), jnp.float32),
            pltpu.VMEM((TBLK * DIM,), jnp.float32),
            pltpu.VMEM((TBLK * DIM,), jnp.float32),
            pltpu.SemaphoreType.DMA,
            pltpu.SemaphoreType.DMA,
            pltpu.SemaphoreType.DMA,
            pltpu.SemaphoreType.DMA,
        ],
    )
    def k(projT_hbm, tail_hbm, flat_hbm, tb0, tb1, rb0, rb1, *sems):
        tbuf = (tb0, tb1)
        rbuf = (rb0, rb1)
        isem = sems[:2]
        osem = sems[2:]
        c = lax.axis_index("c")
        s = lax.axis_index("s")
        wid = s * NC + c
        iota = lax.iota(jnp.int32, LANES)

        def transpose_cols(tb, rb, ncols):
            unroll = 1

            def body(g, carry):
                for u in range(unroll):
                    i0 = (g * unroll + u) * LANES
                    base = (lax.broadcast(i0, (LANES,)) + iota) * DIM
                    for d in range(DIM):
                        v = tbuf[tb][d, pl.ds(i0, LANES)]
                        plsc.store_scatter(
                            rbuf[rb], [base + jnp.int32(d)], v
                        )
                return carry

            lax.fori_loop(0, ncols // (unroll * LANES), body, 0)

        def src_dst(t):
            blk = wid + t * NW
            b0 = pl.multiple_of(blk * TBLK, 128)
            src = projT_hbm.at[:, pl.ds(b0, TBLK)]
            dst = flat_hbm.at[pl.ds(b0 * DIM, TBLK * DIM)]
            return src, dst

        def guard(t):
            return wid + t * NW < nblk

        @pl.when(guard(0))
        def _():
            src, _ = src_dst(0)
            pltpu.async_copy(src, tbuf[0], isem[0])

        for t in range(per_w):
            p = t % 2

            @pl.when(guard(t))
            def _(t=t, p=p):
                src, _ = src_dst(t)
                pltpu.make_async_copy(src, tbuf[p], isem[p]).wait()

            if t + 1 < per_w:
                @pl.when(guard(t + 1))
                def _(t=t, p=p):
                    src, _ = src_dst(t + 1)
                    pltpu.async_copy(src, tbuf[1 - p], isem[1 - p])

            if t - 2 >= 0:
                @pl.when(guard(t))
                def _(t=t, p=p):
                    _, dst = src_dst(t - 2)
                    pltpu.make_async_copy(rbuf[p], dst, osem[p]).wait()

            @pl.when(guard(t))
            def _(t=t, p=p):
                transpose_cols(p, p, TBLK)
                _, dst = src_dst(t)
                pltpu.async_copy(rbuf[p], dst, osem[p])

        for t in range(max(0, per_w - 3), per_w):
            @pl.when(jnp.logical_and(guard(t), jnp.logical_not(guard(t + 2))))
            def _(t=t):
                _, dst = src_dst(t)
                pltpu.make_async_copy(rbuf[t % 2], dst, osem[t % 2]).wait()

        if tail_b:
            @pl.when(wid == NW - 1)
            def _():
                b0 = pl.multiple_of(nblk * TBLK, 128)
                pltpu.sync_copy(
                    projT_hbm.at[:, pl.ds(b0, tail_b)],
                    tbuf[0].at[:, pl.ds(0, tail_b)],
                )
                transpose_cols(0, 0, tail_b)
                pltpu.sync_copy(
                    rbuf[0].at[pl.ds(0, tail_b * DIM)],
                    flat_hbm.at[pl.ds(nblk * TBLK * DIM, tail_b * DIM)],
                )

        @pl.when(wid == NW - 2)
        def _():
            pltpu.sync_copy(tail_hbm, tbuf[0].at[:, pl.ds(0, 128)])
            transpose_cols(0, 0, 128)
            pltpu.sync_copy(
                rbuf[0].at[pl.ds(0, 128 * DIM)],
                flat_hbm.at[pl.ds((vocab - 128) * DIM, 128 * DIM)],
            )

    return k


@functools.lru_cache(maxsize=None)
def _softalign_sc(batch, fields):
    n_rows = batch * fields
    bpw = n_rows // NW
    nch = bpw // CHUNK
    cpf = batch // CHUNK
    mesh = plsc.VectorSubcoreMesh(core_axis_name="c", subcore_axis_name="s")

    @functools.partial(
        pl.kernel,
        mesh=mesh,
        compiler_params=pltpu.CompilerParams(
            needs_layout_passes=False, use_tc_tiling_on_sc=False
        ),
        out_type=jax.ShapeDtypeStruct((fields, DIM, batch), jnp.float32),
        scratch_types=(
            [
                pltpu.VMEM((nch, CHUNK), jnp.int32),
                pltpu.VMEM((NG, CHUNK, DIM), jnp.float32),
                pltpu.VMEM((NO, DIM, CHUNK), jnp.float32),
            ]
            + [pltpu.SemaphoreType.DMA] * (NG + NO)
        ),
    )
    def k(idxT_hbm, proj_hbm, outT_hbm, idx_v, gbuf_v, obuf_v, *sems):
        gsem = sems[:NG]
        osem = sems[NG:]

        c = lax.axis_index("c")
        s = lax.axis_index("s")
        wid = s * NC + c
        jc0 = wid * nch

        for t in range(nch):
            jc = jc0 + t
            f = jc // cpf
            b0 = (jc % cpf) * CHUNK
            pltpu.sync_copy(idxT_hbm.at[f, pl.ds(b0, CHUNK)], idx_v.at[t])

        def fire_gather(t):
            b = t % NG
            return pltpu.async_copy(
                proj_hbm.at[idx_v.at[t]], gbuf_v.at[b], gsem[b]
            )

        pending_g = {0: fire_gather(0)}
        pending_o = {}
        iota = lax.iota(jnp.int32, LANES)

        for j in range(nch):
            b = j % NG
            o = j % NO
            pending_g.pop(j).wait()
            if j + 1 < nch:
                pending_g[j + 1] = fire_gather(j + 1)
            if j - NO >= 0:
                pending_o.pop(j - NO).wait()

            def body(g, carry):
                for u in range(2):
                    g0 = g * 2 + u
                    rows = lax.broadcast(g0 * LANES, (LANES,)) + iota
                    es = []
                    for col in range(DIM):
                        cols = jnp.full((LANES,), col, jnp.int32)
                        v = plsc.load_gather(gbuf_v.at[b], [rows, cols])
                        es.append(jnp.exp(v))
                    acc = es
                    while len(acc) > 1:
                        acc = [
                            acc[i] + acc[i + 1] for i in range(0, len(acc), 2)
                        ]
                    r = 1.0 / acc[0]
                    for col in range(DIM):
                        obuf_v[o, col, pl.ds(g0 * LANES, LANES)] = es[col] * r
                return carry

            lax.fori_loop(0, CHUNK // LANES // 2, body, 0)

            jc = jc0 + j
            f = jc // cpf
            b0 = (jc % cpf) * CHUNK
            pending_o[j] = pltpu.async_copy(
                obuf_v.at[o],
                outT_hbm.at[f, :, pl.ds(b0, CHUNK)],
                osem[o],
            )

        for j in sorted(pending_o):
            pending_o[j].wait()

    return k


def kernel(input, proj):
    batch, fields = input.shape
    vocab = proj.shape[0]
    projT = proj.T
    tail128 = lax.slice(projT, (0, vocab - 128), (DIM, vocab))
    flat = _transpose_sc(vocab)(projT, tail128)
    proj_rm = flat.reshape(vocab, DIM)
    outT = _softalign_sc(batch, fields)(input.T.astype(jnp.int32), proj_rm)
    return outT.transpose(2, 0, 1)


# speedup vs baseline: 1.0101x; 1.0090x over previous
import functools

import jax
import jax.numpy as jnp
from jax import lax
from jax.experimental import pallas as pl
from jax.experimental.pallas import tpu as pltpu
from jax.experimental.pallas import tpu_sc as plsc

DIM = 16
LANES = 16
NC = 2
NS = 16
NW = NC * NS
CHUNK = 1024
NG = 2
NO = 2
TBLK = 1024
TUNROLL = 8


@functools.lru_cache(maxsize=None)
def _transpose_sc(vocab):
    nblk = vocab // TBLK
    rem = vocab - nblk * TBLK
    tail_b = (rem // 128) * 128
    per_w = (nblk + NW - 1) // NW
    mesh = plsc.VectorSubcoreMesh(core_axis_name="c", subcore_axis_name="s")

    @functools.partial(
        pl.kernel,
        mesh=mesh,
        compiler_params=pltpu.CompilerParams(
            needs_layout_passes=False, use_tc_tiling_on_sc=True
        ),
        out_type=jax.ShapeDtypeStruct((vocab * DIM,), jnp.float32),
        scratch_types=[
            pltpu.VMEM((DIM, TBLK), jnp.float32),
            pltpu.VMEM((DIM, TBLK), jnp.float32),
            pltpu.VMEM((TBLK * DIM,), jnp.float32),
            pltpu.VMEM((TBLK * DIM,), jnp.float32),
            pltpu.SemaphoreType.DMA,
            pltpu.SemaphoreType.DMA,
            pltpu.SemaphoreType.DMA,
            pltpu.SemaphoreType.DMA,
        ],
    )
    def k(projT_hbm, tail_hbm, flat_hbm, tb0, tb1, rb0, rb1, *sems):
        tbuf = (tb0, tb1)
        rbuf = (rb0, rb1)
        isem = sems[:2]
        osem = sems[2:]
        c = lax.axis_index("c")
        s = lax.axis_index("s")
        wid = s * NC + c
        iota = lax.iota(jnp.int32, LANES)

        def transpose_cols(tb, rb, ncols):
            def body(g, base):
                i0 = g * LANES
                for d in range(DIM):
                    v = tbuf[tb][d, pl.ds(i0, LANES)]
                    plsc.store_scatter(rbuf[rb], [base + jnp.int32(d)], v)
                return base + jnp.int32(LANES * DIM)

            lax.fori_loop(0, ncols // LANES, body, iota * DIM)

        def src_dst(t):
            blk = wid + t * NW
            b0 = pl.multiple_of(blk * TBLK, 128)
            src = projT_hbm.at[:, pl.ds(b0, TBLK)]
            dst = flat_hbm.at[pl.ds(b0 * DIM, TBLK * DIM)]
            return src, dst

        def guard(t):
            return wid + t * NW < nblk

        @pl.when(guard(0))
        def _():
            src, _ = src_dst(0)
            pltpu.async_copy(src, tbuf[0], isem[0])

        for t in range(per_w):
            p = t % 2

            @pl.when(guard(t))
            def _(t=t, p=p):
                src, _ = src_dst(t)
                pltpu.make_async_copy(src, tbuf[p], isem[p]).wait()

            if t + 1 < per_w:
                @pl.when(guard(t + 1))
                def _(t=t, p=p):
                    src, _ = src_dst(t + 1)
                    pltpu.async_copy(src, tbuf[1 - p], isem[1 - p])

            if t - 2 >= 0:
                @pl.when(guard(t))
                def _(t=t, p=p):
                    _, dst = src_dst(t - 2)
                    pltpu.make_async_copy(rbuf[p], dst, osem[p]).wait()

            @pl.when(guard(t))
            def _(t=t, p=p):
                transpose_cols(p, p, TBLK)
                _, dst = src_dst(t)
                pltpu.async_copy(rbuf[p], dst, osem[p])

        for t in range(max(0, per_w - 3), per_w):
            @pl.when(jnp.logical_and(guard(t), jnp.logical_not(guard(t + 2))))
            def _(t=t):
                _, dst = src_dst(t)
                pltpu.make_async_copy(rbuf[t % 2], dst, osem[t % 2]).wait()

        if tail_b:
            @pl.when(wid == NW - 1)
            def _():
                b0 = pl.multiple_of(nblk * TBLK, 128)
                pltpu.sync_copy(
                    projT_hbm.at[:, pl.ds(b0, tail_b)],
                    tbuf[0].at[:, pl.ds(0, tail_b)],
                )
                transpose_cols(0, 0, tail_b)
                pltpu.sync_copy(
                    rbuf[0].at[pl.ds(0, tail_b * DIM)],
                    flat_hbm.at[pl.ds(nblk * TBLK * DIM, tail_b * DIM)],
                )

        @pl.when(wid == NW - 2)
        def _():
            pltpu.sync_copy(tail_hbm, tbuf[0].at[:, pl.ds(0, 128)])
            transpose_cols(0, 0, 128)
            pltpu.sync_copy(
                rbuf[0].at[pl.ds(0, 128 * DIM)],
                flat_hbm.at[pl.ds((vocab - 128) * DIM, 128 * DIM)],
            )

    return k


@functools.lru_cache(maxsize=None)
def _softalign_sc(batch, fields):
    n_rows = batch * fields
    bpw = n_rows // NW
    nch = bpw // CHUNK
    cpf = batch // CHUNK
    mesh = plsc.VectorSubcoreMesh(core_axis_name="c", subcore_axis_name="s")

    @functools.partial(
        pl.kernel,
        mesh=mesh,
        compiler_params=pltpu.CompilerParams(
            needs_layout_passes=False, use_tc_tiling_on_sc=False
        ),
        out_type=jax.ShapeDtypeStruct((fields, DIM, batch), jnp.float32),
        scratch_types=(
            [
                pltpu.VMEM((nch, CHUNK), jnp.int32),
                pltpu.VMEM((NG, CHUNK, DIM), jnp.float32),
                pltpu.VMEM((NO, DIM, CHUNK), jnp.float32),
            ]
            + [pltpu.SemaphoreType.DMA] * (NG + NO)
        ),
    )
    def k(idxT_hbm, proj_hbm, outT_hbm, idx_v, gbuf_v, obuf_v, *sems):
        gsem = sems[:NG]
        osem = sems[NG:]

        c = lax.axis_index("c")
        s = lax.axis_index("s")
        wid = s * NC + c
        jc0 = wid * nch

        for t in range(nch):
            jc = jc0 + t
            f = jc // cpf
            b0 = (jc % cpf) * CHUNK
            pltpu.sync_copy(idxT_hbm.at[f, pl.ds(b0, CHUNK)], idx_v.at[t])

        def fire_gather(t):
            b = t % NG
            return pltpu.async_copy(
                proj_hbm.at[idx_v.at[t]], gbuf_v.at[b], gsem[b]
            )

        pending_g = {0: fire_gather(0)}
        pending_o = {}
        iota = lax.iota(jnp.int32, LANES)

        for j in range(nch):
            b = j % NG
            o = j % NO
            pending_g.pop(j).wait()
            if j + 1 < nch:
                pending_g[j + 1] = fire_gather(j + 1)
            if j - NO >= 0:
                pending_o.pop(j - NO).wait()

            def body(g, rows):
                es = []
                for col in range(DIM):
                    cols = jnp.full((LANES,), col, jnp.int32)
                    v = plsc.load_gather(gbuf_v.at[b], [rows, cols])
                    es.append(jnp.exp(v))
                acc = es
                while len(acc) > 1:
                    acc = [acc[i] + acc[i + 1] for i in range(0, len(acc), 2)]
                r = 1.0 / acc[0]
                for col in range(DIM):
                    obuf_v[o, col, pl.ds(g * LANES, LANES)] = es[col] * r
                return rows + jnp.int32(LANES)

            lax.fori_loop(0, CHUNK // LANES, body, iota)

            jc = jc0 + j
            f = jc // cpf
            b0 = (jc % cpf) * CHUNK
            pending_o[j] = pltpu.async_copy(
                obuf_v.at[o],
                outT_hbm.at[f, :, pl.ds(b0, CHUNK)],
                osem[o],
            )

        for j in sorted(pending_o):
            pending_o[j].wait()

    return k


def kernel(input, proj):
    batch, fields = input.shape
    vocab = proj.shape[0]
    projT = proj.T
    tail128 = lax.slice(projT, (0, vocab - 128), (DIM, vocab))
    flat = _transpose_sc(vocab)(projT, tail128)
    proj_rm = flat.reshape(vocab, DIM)
    outT = _softalign_sc(batch, fields)(input.T.astype(jnp.int32), proj_rm)
    return outT.transpose(2, 0, 1)
